# Initial kernel scaffold; baseline (speedup 1.0000x reference)
import jax
import jax.numpy as jnp
from jax.experimental import pallas as pl


def kernel(relative_position_bias_table, relative_position_index):
    raise NotImplementedError("write your pallas kernel here")


# speedup vs baseline: 14.2166x; 14.2166x over previous
import functools

import jax
import jax.numpy as jnp
from jax import lax
from jax.experimental import pallas as pl
from jax.experimental.pallas import tpu as pltpu
from jax.experimental.pallas import tpu_sc as plsc

H = 32
W = 32
NUM_HEADS = 16
TOKENS = H * W
NUM_REL = (2 * H - 1) * (2 * W - 1)
ROWS_PER_STRIP = 8
COL_PAD = 4096

_mesh = plsc.VectorSubcoreMesh(core_axis_name="c", subcore_axis_name="s")


@functools.partial(
    pl.kernel,
    mesh=_mesh,
    out_type=jax.ShapeDtypeStruct((NUM_HEADS, TOKENS, TOKENS), jnp.float32),
    compiler_params=pltpu.CompilerParams(needs_layout_passes=False),
    scratch_types=[
        pltpu.VMEM((NUM_REL * NUM_HEADS,), jnp.float32),
        pltpu.VMEM((COL_PAD,), jnp.float32),
        pltpu.VMEM((ROWS_PER_STRIP, TOKENS), jnp.int32),
        pltpu.VMEM((ROWS_PER_STRIP, TOKENS), jnp.float32),
    ],
)
def _sc_bias_gather(table_hbm, idx_hbm, out_hbm, tab_v, col_v, idx_v, out_v):
    h = lax.axis_index("s")
    half = lax.axis_index("c")

    pltpu.sync_copy(table_hbm, tab_v)
    lanes = lax.iota(jnp.int32, 16)
    for k in range(COL_PAD // 16):
        rows = jnp.minimum(k * 16 + lanes, NUM_REL - 1)
        col_v[pl.ds(k * 16, 16)] = plsc.load_gather(
            tab_v, [rows * NUM_HEADS + h])

    n_strips = (TOKENS // 2) // ROWS_PER_STRIP

    def strip_body(t, carry):
        r0 = half * (TOKENS // 2) + t * ROWS_PER_STRIP
        pltpu.sync_copy(idx_hbm.at[pl.ds(r0, ROWS_PER_STRIP), :], idx_v)
        for r in range(ROWS_PER_STRIP):
            for k in range(TOKENS // 16):
                iv = idx_v[r, pl.ds(k * 16, 16)]
                out_v[r, pl.ds(k * 16, 16)] = plsc.load_gather(col_v, [iv])
        pltpu.sync_copy(out_v, out_hbm.at[h, pl.ds(r0, ROWS_PER_STRIP), :])
        return carry

    lax.fori_loop(0, n_strips, strip_body, 0)


def kernel(relative_position_bias_table, relative_position_index):
    return _sc_bias_gather(relative_position_bias_table.reshape(-1),
                           relative_position_index)
